# Initial kernel scaffold; baseline (speedup 1.0000x reference)
import jax
import jax.numpy as jnp
from jax.experimental import pallas as pl


def kernel(x, senders, receivers, weight, bias):
    raise NotImplementedError("write your pallas kernel here")


# speedup vs baseline: 4.3069x; 4.3069x over previous
import functools

import jax
import jax.numpy as jnp
from jax import lax
from jax.experimental import pallas as pl
from jax.experimental.pallas import tpu as pltpu
from jax.experimental.pallas import tpu_sc as plsc

N_NODES = 10000
N_EDGES = 320000
D = 128

NC = 2
NS = 16
NW = NC * NS
CHUNK = 128
NCH = 80
EPT = NCH * CHUNK
E_PAD = NW * EPT
N_PAD = 10240
RPT = N_PAD // NS
PAD_IDX = N_NODES

_mesh = plsc.VectorSubcoreMesh(core_axis_name="c", subcore_axis_name="s")


@functools.partial(
    pl.kernel,
    out_type=[
        jax.ShapeDtypeStruct((NC, N_PAD), jnp.float32),
        jax.ShapeDtypeStruct((NC, N_PAD), jnp.float32),
    ],
    mesh=_mesh,
    scratch_types=[
        pltpu.VMEM((NCH, CHUNK), jnp.int32),
        pltpu.VMEM((NCH, CHUNK), jnp.int32),
        pltpu.VMEM((CHUNK,), jnp.float32),
        pltpu.VMEM((RPT,), jnp.float32),
        pltpu.VMEM_SHARED((N_PAD,), jnp.float32),
        pltpu.VMEM_SHARED((N_PAD,), jnp.float32),
    ],
)
def _sc_degrees(s_hbm, r_hbm, sd_out, rd_out, sidx, ridx, ones, zbuf, sdeg, rdeg):
    c = lax.axis_index("c")
    s = lax.axis_index("s")
    wid = c * NS + s

    pltpu.sync_copy(s_hbm.at[wid], sidx)
    pltpu.sync_copy(r_hbm.at[wid], ridx)

    def fill(i, _):
        ones[pl.ds(i * 16, 16)] = jnp.ones((16,), jnp.float32)
        return _

    lax.fori_loop(0, CHUNK // 16, fill, None)

    def zfill(i, _):
        zbuf[pl.ds(i * 16, 16)] = jnp.zeros((16,), jnp.float32)
        return _

    lax.fori_loop(0, RPT // 16, zfill, None)

    pltpu.sync_copy(zbuf, sdeg.at[pl.ds(s * RPT, RPT)])
    pltpu.sync_copy(zbuf, rdeg.at[pl.ds(s * RPT, RPT)])
    plsc.subcore_barrier()

    def body(ch, _):
        pltpu.sync_copy(ones, sdeg.at[sidx.at[ch]], add=True)
        pltpu.sync_copy(ones, rdeg.at[ridx.at[ch]], add=True)
        return _

    lax.fori_loop(0, NCH, body, None)
    plsc.subcore_barrier()

    pltpu.sync_copy(sdeg.at[pl.ds(s * RPT, RPT)], sd_out.at[c, pl.ds(s * RPT, RPT)])
    pltpu.sync_copy(rdeg.at[pl.ds(s * RPT, RPT)], rd_out.at[c, pl.ds(s * RPT, RPT)])


@functools.partial(
    pl.kernel,
    out_type=jax.ShapeDtypeStruct((NC, N_PAD, D), jnp.float32),
    mesh=_mesh,
    scratch_types=[
        pltpu.VMEM((NCH, CHUNK), jnp.int32),
        pltpu.VMEM((NCH, CHUNK), jnp.int32),
        pltpu.VMEM((1, CHUNK, D), jnp.float32),
        pltpu.VMEM_SHARED((N_PAD, D), jnp.float32),
        pltpu.SemaphoreType.DMA,
    ],
)
def _sc_aggregate(g_hbm, s_hbm, r_hbm, p_out, sidx, ridx, rows, acc, sem):
    c = lax.axis_index("c")
    s = lax.axis_index("s")
    wid = c * NS + s

    pltpu.sync_copy(s_hbm.at[wid], sidx)
    pltpu.sync_copy(r_hbm.at[wid], ridx)

    def zfill(i, _):
        for j in range(D // 16):
            rows[0, i, pl.ds(j * 16, 16)] = jnp.zeros((16,), jnp.float32)
        return _

    lax.fori_loop(0, CHUNK, zfill, None)
    for k in range(RPT // CHUNK):
        pltpu.sync_copy(rows.at[0], acc.at[pl.ds(s * RPT + k * CHUNK, CHUNK)])
    plsc.subcore_barrier()

    def body(ch, _):
        pltpu.async_copy(g_hbm.at[sidx.at[ch]], rows.at[0], sem).wait()
        pltpu.sync_copy(rows.at[0], acc.at[ridx.at[ch]], add=True)
        return _

    lax.fori_loop(0, NCH, body, None)
    plsc.subcore_barrier()

    pltpu.sync_copy(acc.at[pl.ds(s * RPT, RPT)], p_out.at[c, pl.ds(s * RPT, RPT)])


def _tc_transform_body(x_ref, w_ref, b_ref, sd_ref, g_ref):
    h = jnp.dot(x_ref[...], w_ref[...], preferred_element_type=jnp.float32)
    h = h + b_ref[...]
    sd = sd_ref[0, :] + sd_ref[1, :]
    g_ref[...] = h * lax.rsqrt(jnp.maximum(sd, 1.0))[:, None]


def _tc_finalize_body(p_ref, rd_ref, o_ref):
    acc = p_ref[0] + p_ref[1]
    rd = rd_ref[0, :] + rd_ref[1, :]
    o_ref[...] = acc * lax.rsqrt(jnp.maximum(rd, 1.0))[:, None]


_ROWS_BLK = 1280


def kernel(x, senders, receivers, weight, bias):
    x_pad = jnp.pad(x, ((0, N_PAD - N_NODES), (0, 0)))
    pad = jnp.full((E_PAD - N_EDGES,), PAD_IDX, jnp.int32)
    s_pad = jnp.concatenate([senders, pad]).reshape(NW, NCH, CHUNK)
    r_pad = jnp.concatenate([receivers, pad]).reshape(NW, NCH, CHUNK)

    sd_p, rd_p = _sc_degrees(s_pad, r_pad)

    g = pl.pallas_call(
        _tc_transform_body,
        grid=(N_PAD // _ROWS_BLK,),
        in_specs=[
            pl.BlockSpec((_ROWS_BLK, D), lambda i: (i, 0)),
            pl.BlockSpec((D, D), lambda i: (0, 0)),
            pl.BlockSpec((1, D), lambda i: (0, 0)),
            pl.BlockSpec((NC, _ROWS_BLK), lambda i: (0, i)),
        ],
        out_specs=pl.BlockSpec((_ROWS_BLK, D), lambda i: (i, 0)),
        out_shape=jax.ShapeDtypeStruct((N_PAD, D), jnp.float32),
    )(x_pad, weight, bias.reshape(1, D), sd_p)

    p = _sc_aggregate(g, s_pad, r_pad)

    out = pl.pallas_call(
        _tc_finalize_body,
        grid=(N_PAD // _ROWS_BLK,),
        in_specs=[
            pl.BlockSpec((NC, _ROWS_BLK, D), lambda i: (0, i, 0)),
            pl.BlockSpec((NC, _ROWS_BLK), lambda i: (0, i)),
        ],
        out_specs=pl.BlockSpec((_ROWS_BLK, D), lambda i: (i, 0)),
        out_shape=jax.ShapeDtypeStruct((N_NODES, D), jnp.float32),
    )(p, rd_p)
    return out


# speedup vs baseline: 4.8252x; 1.1204x over previous
import functools

import jax
import jax.numpy as jnp
from jax import lax
from jax.experimental import pallas as pl
from jax.experimental.pallas import tpu as pltpu
from jax.experimental.pallas import tpu_sc as plsc

N_NODES = 10000
N_EDGES = 320000
D = 128

NC = 2
NS = 16
NW = NC * NS
CHUNK = 128
NCH = 80
NCHG = 16
NGRP = NCH // NCHG
EPT = NCH * CHUNK
E_PAD = NW * EPT
N_PAD = 10240
RPT = N_PAD // NS
PAD_IDX = N_NODES

_mesh = plsc.VectorSubcoreMesh(core_axis_name="c", subcore_axis_name="s")


@functools.partial(
    pl.kernel,
    out_type=[
        jax.ShapeDtypeStruct((NC, N_PAD), jnp.float32),
        jax.ShapeDtypeStruct((NC, N_PAD), jnp.float32),
    ],
    mesh=_mesh,
    scratch_types=[
        pltpu.VMEM((NCH, CHUNK), jnp.int32),
        pltpu.VMEM((NCH, CHUNK), jnp.int32),
        pltpu.VMEM((CHUNK,), jnp.float32),
        pltpu.VMEM((RPT,), jnp.float32),
        pltpu.VMEM_SHARED((N_PAD,), jnp.float32),
        pltpu.VMEM_SHARED((N_PAD,), jnp.float32),
    ],
)
def _sc_degrees(s_hbm, r_hbm, sd_out, rd_out, sidx, ridx, ones, zbuf, sdeg, rdeg):
    c = lax.axis_index("c")
    s = lax.axis_index("s")
    wid = c * NS + s

    pltpu.sync_copy(s_hbm.at[wid], sidx)
    pltpu.sync_copy(r_hbm.at[wid], ridx)

    def fill(i, _):
        ones[pl.ds(i * 16, 16)] = jnp.ones((16,), jnp.float32)
        return _

    lax.fori_loop(0, CHUNK // 16, fill, None)

    def zfill(i, _):
        zbuf[pl.ds(i * 16, 16)] = jnp.zeros((16,), jnp.float32)
        return _

    lax.fori_loop(0, RPT // 16, zfill, None)

    pltpu.sync_copy(zbuf, sdeg.at[pl.ds(s * RPT, RPT)])
    pltpu.sync_copy(zbuf, rdeg.at[pl.ds(s * RPT, RPT)])
    plsc.subcore_barrier()

    def body(ch, _):
        pltpu.sync_copy(ones, sdeg.at[sidx.at[ch]], add=True)
        pltpu.sync_copy(ones, rdeg.at[ridx.at[ch]], add=True)
        return _

    lax.fori_loop(0, NCH, body, None)
    plsc.subcore_barrier()

    pltpu.sync_copy(sdeg.at[pl.ds(s * RPT, RPT)], sd_out.at[c, pl.ds(s * RPT, RPT)])
    pltpu.sync_copy(rdeg.at[pl.ds(s * RPT, RPT)], rd_out.at[c, pl.ds(s * RPT, RPT)])


@functools.partial(
    pl.kernel,
    out_type=jax.ShapeDtypeStruct((NC, N_PAD, D), jnp.float32),
    mesh=_mesh,
    scratch_types=[
        pltpu.VMEM((NCHG, CHUNK), jnp.int32),
        pltpu.VMEM((NCHG, CHUNK), jnp.int32),
        pltpu.VMEM((2, CHUNK, D), jnp.float32),
        pltpu.VMEM_SHARED((N_PAD, D), jnp.float32),
        pltpu.SemaphoreType.DMA,
        pltpu.SemaphoreType.DMA,
    ],
)
def _sc_aggregate(g_hbm, s_hbm, r_hbm, p_out, sidx, ridx, rows, acc, sem0, sem1):
    c = lax.axis_index("c")
    s = lax.axis_index("s")
    wid = c * NS + s
    sems = (sem0, sem1)

    def zfill(i, _):
        for j in range(D // 16):
            rows[0, i, pl.ds(j * 16, 16)] = jnp.zeros((16,), jnp.float32)
        return _

    lax.fori_loop(0, CHUNK, zfill, None)
    for k in range(RPT // CHUNK):
        pltpu.sync_copy(rows.at[0], acc.at[pl.ds(s * RPT + k * CHUNK, CHUNK)])
    rem = RPT - (RPT // CHUNK) * CHUNK
    if rem:
        pltpu.sync_copy(
            rows.at[0, pl.ds(0, rem)],
            acc.at[pl.ds(s * RPT + (RPT // CHUNK) * CHUNK, rem)],
        )
    plsc.subcore_barrier()

    for q in range(NGRP):
        pltpu.sync_copy(s_hbm.at[wid, pl.ds(q * NCHG, NCHG)], sidx)
        pltpu.sync_copy(r_hbm.at[wid, pl.ds(q * NCHG, NCHG)], ridx)
        pltpu.async_copy(g_hbm.at[sidx.at[0]], rows.at[0], sem0)

        def body(i, _):
            for b in range(2):
                ch = i * 2 + b
                nxt = ch + 1

                @pl.when(nxt < NCHG)
                def _start():
                    pltpu.async_copy(
                        g_hbm.at[sidx.at[nxt]], rows.at[1 - b], sems[1 - b]
                    )

                pltpu.make_async_copy(g_hbm.at[sidx.at[ch]], rows.at[b], sems[b]).wait()
                pltpu.sync_copy(rows.at[b], acc.at[ridx.at[ch]], add=True)
            return _

        lax.fori_loop(0, NCHG // 2, body, None)
    plsc.subcore_barrier()

    pltpu.sync_copy(acc.at[pl.ds(s * RPT, RPT)], p_out.at[c, pl.ds(s * RPT, RPT)])


def _tc_transform_body(x_ref, w_ref, b_ref, sd_ref, g_ref):
    h = jnp.dot(x_ref[...], w_ref[...], preferred_element_type=jnp.float32)
    h = h + b_ref[...]
    sd = sd_ref[0, :] + sd_ref[1, :]
    g_ref[...] = h * lax.rsqrt(jnp.maximum(sd, 1.0))[:, None]


def _tc_finalize_body(p_ref, rd_ref, o_ref):
    acc = p_ref[0] + p_ref[1]
    rd = rd_ref[0, :] + rd_ref[1, :]
    o_ref[...] = acc * lax.rsqrt(jnp.maximum(rd, 1.0))[:, None]


_ROWS_BLK = 1280


def kernel(x, senders, receivers, weight, bias):
    x_pad = jnp.pad(x, ((0, N_PAD - N_NODES), (0, 0)))
    pad = jnp.full((E_PAD - N_EDGES,), PAD_IDX, jnp.int32)
    s_pad = jnp.concatenate([senders, pad]).reshape(NW, NCH, CHUNK)
    r_pad = jnp.concatenate([receivers, pad]).reshape(NW, NCH, CHUNK)

    sd_p, rd_p = _sc_degrees(s_pad, r_pad)

    g = pl.pallas_call(
        _tc_transform_body,
        grid=(N_PAD // _ROWS_BLK,),
        in_specs=[
            pl.BlockSpec((_ROWS_BLK, D), lambda i: (i, 0)),
            pl.BlockSpec((D, D), lambda i: (0, 0)),
            pl.BlockSpec((1, D), lambda i: (0, 0)),
            pl.BlockSpec((NC, _ROWS_BLK), lambda i: (0, i)),
        ],
        out_specs=pl.BlockSpec((_ROWS_BLK, D), lambda i: (i, 0)),
        out_shape=jax.ShapeDtypeStruct((N_PAD, D), jnp.float32),
    )(x_pad, weight, bias.reshape(1, D), sd_p)

    p = _sc_aggregate(g, s_pad, r_pad)

    out = pl.pallas_call(
        _tc_finalize_body,
        grid=(N_PAD // _ROWS_BLK,),
        in_specs=[
            pl.BlockSpec((NC, _ROWS_BLK, D), lambda i: (0, i, 0)),
            pl.BlockSpec((NC, _ROWS_BLK), lambda i: (0, i)),
        ],
        out_specs=pl.BlockSpec((_ROWS_BLK, D), lambda i: (i, 0)),
        out_shape=jax.ShapeDtypeStruct((N_NODES, D), jnp.float32),
    )(p, rd_p)
    return out


# speedup vs baseline: 12.8337x; 2.6597x over previous
import functools

import jax
import jax.numpy as jnp
from jax import lax
from jax.experimental import pallas as pl
from jax.experimental.pallas import tpu as pltpu
from jax.experimental.pallas import tpu_sc as plsc

N_NODES = 10000
N_EDGES = 320000
D = 128

NC = 2
NS = 16
NW = NC * NS
CHUNK = 128
NCH = 80
NCHG = 16
NGRP = NCH // NCHG
EPT = NCH * CHUNK
E_PAD = NW * EPT
N_PAD = 10240
RPT = N_PAD // NS
PAD_IDX = N_NODES

_mesh = plsc.VectorSubcoreMesh(core_axis_name="c", subcore_axis_name="s")


@functools.partial(
    pl.kernel,
    out_type=[
        jax.ShapeDtypeStruct((NC, N_PAD), jnp.float32),
        jax.ShapeDtypeStruct((NC, N_PAD), jnp.float32),
    ],
    mesh=_mesh,
    scratch_types=[
        pltpu.VMEM((NCH, CHUNK), jnp.int32),
        pltpu.VMEM((NCH, CHUNK), jnp.int32),
        pltpu.VMEM((CHUNK,), jnp.float32),
        pltpu.VMEM((RPT,), jnp.float32),
        pltpu.VMEM_SHARED((N_PAD,), jnp.float32),
        pltpu.VMEM_SHARED((N_PAD,), jnp.float32),
    ],
)
def _sc_degrees(s_hbm, r_hbm, sd_out, rd_out, sidx, ridx, ones, zbuf, sdeg, rdeg):
    c = lax.axis_index("c")
    s = lax.axis_index("s")
    wid = c * NS + s

    pltpu.sync_copy(s_hbm.at[wid], sidx)
    pltpu.sync_copy(r_hbm.at[wid], ridx)

    def fill(i, _):
        ones[pl.ds(i * 16, 16)] = jnp.ones((16,), jnp.float32)
        return _

    lax.fori_loop(0, CHUNK // 16, fill, None)

    def zfill(i, _):
        zbuf[pl.ds(i * 16, 16)] = jnp.zeros((16,), jnp.float32)
        return _

    lax.fori_loop(0, RPT // 16, zfill, None)

    pltpu.sync_copy(zbuf, sdeg.at[pl.ds(s * RPT, RPT)])
    pltpu.sync_copy(zbuf, rdeg.at[pl.ds(s * RPT, RPT)])
    plsc.subcore_barrier()

    def body(ch, _):
        pltpu.sync_copy(ones, sdeg.at[sidx.at[ch]], add=True)
        pltpu.sync_copy(ones, rdeg.at[ridx.at[ch]], add=True)
        return _

    lax.fori_loop(0, NCH, body, None)
    plsc.subcore_barrier()

    pltpu.sync_copy(sdeg.at[pl.ds(s * RPT, RPT)], sd_out.at[c, pl.ds(s * RPT, RPT)])
    pltpu.sync_copy(rdeg.at[pl.ds(s * RPT, RPT)], rd_out.at[c, pl.ds(s * RPT, RPT)])


@functools.partial(
    pl.kernel,
    out_type=jax.ShapeDtypeStruct((NC, N_PAD, D), jnp.float32),
    mesh=_mesh,
    scratch_types=[
        pltpu.VMEM((NCHG, CHUNK), jnp.int32),
        pltpu.VMEM((NCHG, CHUNK), jnp.int32),
        pltpu.VMEM((2, CHUNK, D), jnp.float32),
        pltpu.VMEM_SHARED((N_PAD, D), jnp.float32),
        pltpu.SemaphoreType.DMA,
        pltpu.SemaphoreType.DMA,
    ],
)
def _sc_aggregate(g_hbm, s_hbm, r_hbm, p_out, sidx, ridx, rows, acc, sem0, sem1):
    c = lax.axis_index("c")
    s = lax.axis_index("s")
    wid = c * NS + s
    sems = (sem0, sem1)

    def zfill(i, _):
        for j in range(D // 16):
            rows[0, i, pl.ds(j * 16, 16)] = jnp.zeros((16,), jnp.float32)
        return _

    lax.fori_loop(0, CHUNK, zfill, None)
    for k in range(RPT // CHUNK):
        pltpu.sync_copy(rows.at[0], acc.at[pl.ds(s * RPT + k * CHUNK, CHUNK)])
    rem = RPT - (RPT // CHUNK) * CHUNK
    if rem:
        pltpu.sync_copy(
            rows.at[0, pl.ds(0, rem)],
            acc.at[pl.ds(s * RPT + (RPT // CHUNK) * CHUNK, rem)],
        )
    plsc.subcore_barrier()

    for q in range(NGRP):
        pltpu.sync_copy(s_hbm.at[wid, pl.ds(q * NCHG, NCHG)], sidx)
        pltpu.sync_copy(r_hbm.at[wid, pl.ds(q * NCHG, NCHG)], ridx)
        pltpu.async_copy(g_hbm.at[sidx.at[0]], rows.at[0], sem0)

        def body(i, _):
            for b in range(2):
                ch = i * 2 + b
                nxt = ch + 1

                @pl.when(nxt < NCHG)
                def _start():
                    pltpu.async_copy(
                        g_hbm.at[sidx.at[nxt]], rows.at[1 - b], sems[1 - b]
                    )

                pltpu.make_async_copy(g_hbm.at[sidx.at[ch]], rows.at[b], sems[b]).wait()
                pltpu.sync_copy(rows.at[b], acc.at[ridx.at[ch]], add=True)
            return _

        lax.fori_loop(0, NCHG // 2, body, None)
    plsc.subcore_barrier()

    pltpu.sync_copy(acc.at[pl.ds(s * RPT, RPT)], p_out.at[c, pl.ds(s * RPT, RPT)])


def _tc_transform_body(x_ref, w_ref, b_ref, sd_ref, g_ref):
    h = jnp.dot(x_ref[...], w_ref[...], preferred_element_type=jnp.float32)
    h = h + b_ref[...]
    sd = sd_ref[0, :] + sd_ref[1, :]
    g_ref[...] = h * lax.rsqrt(jnp.maximum(sd, 1.0))[:, None]


def _tc_finalize_body(p_ref, rd_ref, o_ref):
    acc = p_ref[0] + p_ref[1]
    rd = rd_ref[0, :] + rd_ref[1, :]
    o_ref[...] = acc * lax.rsqrt(jnp.maximum(rd, 1.0))[:, None]


_ROWS_BLK = 1280


def kernel(x, senders, receivers, weight, bias):
    x_pad = jnp.pad(x, ((0, N_PAD - N_NODES), (0, 0)))
    pad = PAD_IDX + jnp.arange(E_PAD - N_EDGES, dtype=jnp.int32) % (N_PAD - N_NODES)
    s_pad = jnp.concatenate([senders, pad]).reshape(NW, NCH, CHUNK)
    r_pad = jnp.concatenate([receivers, pad]).reshape(NW, NCH, CHUNK)

    sd_p, rd_p = _sc_degrees(s_pad, r_pad)

    g = pl.pallas_call(
        _tc_transform_body,
        grid=(N_PAD // _ROWS_BLK,),
        in_specs=[
            pl.BlockSpec((_ROWS_BLK, D), lambda i: (i, 0)),
            pl.BlockSpec((D, D), lambda i: (0, 0)),
            pl.BlockSpec((1, D), lambda i: (0, 0)),
            pl.BlockSpec((NC, _ROWS_BLK), lambda i: (0, i)),
        ],
        out_specs=pl.BlockSpec((_ROWS_BLK, D), lambda i: (i, 0)),
        out_shape=jax.ShapeDtypeStruct((N_PAD, D), jnp.float32),
    )(x_pad, weight, bias.reshape(1, D), sd_p)

    p = _sc_aggregate(g, s_pad, r_pad)

    out = pl.pallas_call(
        _tc_finalize_body,
        grid=(N_PAD // _ROWS_BLK,),
        in_specs=[
            pl.BlockSpec((NC, _ROWS_BLK, D), lambda i: (0, i, 0)),
            pl.BlockSpec((NC, _ROWS_BLK), lambda i: (0, i)),
        ],
        out_specs=pl.BlockSpec((_ROWS_BLK, D), lambda i: (i, 0)),
        out_shape=jax.ShapeDtypeStruct((N_NODES, D), jnp.float32),
    )(p, rd_p)
    return out
